# Initial kernel scaffold; baseline (speedup 1.0000x reference)
import jax
import jax.numpy as jnp
from jax.experimental import pallas as pl


def kernel(x, edge_index, W1, as1, ad1, b1, W2, as2, ad2, b2, W3, as3, ad3, b3):
    raise NotImplementedError("write your pallas kernel here")


# speedup vs baseline: 1.0729x; 1.0729x over previous
import jax
import jax.numpy as jnp
from jax.experimental import pallas as pl


def _mean_pool_kernel(h_ref, o_ref):
    o_ref[...] = jnp.mean(h_ref[...], axis=0, keepdims=True)


def _mean_pool(h):
    n, c = h.shape
    return pl.pallas_call(
        _mean_pool_kernel,
        out_shape=jax.ShapeDtypeStruct((1, c), h.dtype),
    )(h)


def _gat_layer(x, src, dst, W, att_src, att_dst, bias, heads, concat, n):
    C = W.shape[1] // heads
    xp = (x @ W).reshape(n, heads, C)
    alpha_src = jnp.sum(xp * att_src, axis=-1)
    alpha_dst = jnp.sum(xp * att_dst, axis=-1)
    e = alpha_src[src] + alpha_dst[dst]
    e = jax.nn.leaky_relu(e, negative_slope=0.2)
    ex = jnp.exp(e)
    denom = jax.ops.segment_sum(ex, dst, num_segments=n)
    alpha = ex / (denom[dst] + 1e-16)
    msg = xp[src] * alpha[:, :, None]
    out = jax.ops.segment_sum(msg, dst, num_segments=n)
    if concat:
        out = out.reshape(n, heads * C)
    else:
        out = out.mean(axis=1)
    return out + bias


def kernel(x, edge_index, W1, as1, ad1, b1, W2, as2, ad2, b2, W3, as3, ad3, b3):
    n = x.shape[0]
    src = edge_index[0].astype(jnp.int32)
    dst = edge_index[1].astype(jnp.int32)
    h = jax.nn.relu(_gat_layer(x, src, dst, W1, as1, ad1, b1, 8, True, n))
    h = jax.nn.relu(_gat_layer(h, src, dst, W2, as2, ad2, b2, 8, True, n))
    h = _gat_layer(h, src, dst, W3, as3, ad3, b3, 1, False, n)
    return _mean_pool(h)


# speedup vs baseline: 6.3120x; 5.8830x over previous
import functools

import jax
import jax.numpy as jnp
from jax import lax
from jax.experimental import pallas as pl
from jax.experimental.pallas import tpu as pltpu
from jax.experimental.pallas import tpu_sc as plsc

N_PAD = 10240
BN = 1024

N_EDGE = 320000
NT = 32
R = 320
M_MAX = 10880
ECH = 2000
GCH = 128

_MESH = plsc.VectorSubcoreMesh(core_axis_name="c", subcore_axis_name="s")


def _proj_kernel(act_ref, b_ref, w_ref, a_ref, xp_ref, asdt_ref, *, heads, relu):
    a = act_ref[...]
    if relu:
        a = jnp.maximum(a + b_ref[...], 0.0)
    xp = jnp.dot(a, w_ref[...], preferred_element_type=jnp.float32,
                 precision=lax.Precision.HIGHEST)
    asd = jnp.dot(xp, a_ref[...], preferred_element_type=jnp.float32,
                  precision=lax.Precision.HIGHEST)
    asdt_ref[...] = asd.T
    c = xp.shape[1] // heads
    xp_ref[...] = xp.reshape(BN, heads, c).transpose(1, 0, 2)


def _project(act, b_prev, W, A_cat, heads, relu):
    f_in = act.shape[1]
    hc = W.shape[1]
    c = hc // heads
    grid = N_PAD // BN
    return pl.pallas_call(
        functools.partial(_proj_kernel, heads=heads, relu=relu),
        grid=(grid,),
        in_specs=[
            pl.BlockSpec((BN, f_in), lambda i: (i, 0)),
            pl.BlockSpec((1, f_in), lambda i: (0, 0)),
            pl.BlockSpec((f_in, hc), lambda i: (0, 0)),
            pl.BlockSpec((hc, 2 * heads), lambda i: (0, 0)),
        ],
        out_specs=[
            pl.BlockSpec((heads, BN, c), lambda i: (0, i, 0)),
            pl.BlockSpec((2 * heads, BN), lambda i: (0, i)),
        ],
        out_shape=[
            jax.ShapeDtypeStruct((heads, N_PAD, c), jnp.float32),
            jax.ShapeDtypeStruct((2 * heads, N_PAD), jnp.float32),
        ],
    )(act, b_prev.reshape(1, f_in), W, A_cat)


def _pool_kernel(h_ref, b_ref, o_ref):
    o_ref[...] = jnp.sum(h_ref[...], axis=0, keepdims=True) / 10000.0 + b_ref[...]


def _mean_pool(h, b3):
    return pl.pallas_call(
        _pool_kernel,
        out_shape=jax.ShapeDtypeStruct((1, h.shape[1]), jnp.float32),
    )(h, b3.reshape(1, h.shape[1]))


def _att_mat(att):
    _, H, C = att.shape
    eye = jnp.eye(H, dtype=jnp.float32)
    return (att[0][:, :, None] * eye[:, None, :]).reshape(H * C, H)


def _wid():
    return lax.axis_index("s") * 2 + lax.axis_index("c")


def _memset(ref, words, value):
    def body(i, _):
        ref[pl.ds(i * 16, 16)] = jnp.full((16,), value, ref.dtype)
        return 0
    lax.fori_loop(0, words // 16, body, 0)


def _part_body(src_hbm, dst_hbm, sp_hbm, dp_hbm, sbuf, dbuf, slist, dlist):
    wid = _wid()
    base = wid * R
    _memset(slist, M_MAX + 16, 0)
    _memset(dlist, M_MAX + 16, R)

    def chunk(k, off):
        pltpu.sync_copy(src_hbm.at[pl.ds(k * ECH, ECH)], sbuf)
        pltpu.sync_copy(dst_hbm.at[pl.ds(k * ECH, ECH)], dbuf)

        def step(j, off):
            sv = sbuf[pl.ds(j * 16, 16)]
            dl = dbuf[pl.ds(j * 16, 16)] - base
            ok = off <= M_MAX - 16
            m = jnp.logical_and((dl >= 0) & (dl < R), ok)
            offc = jnp.minimum(off, M_MAX - 16)
            pos = plsc.cumsum(m.astype(jnp.int32))
            idx = offc + pos - 1
            plsc.store_scatter(slist, [idx], sv, mask=m)
            plsc.store_scatter(dlist, [idx], dl, mask=m)
            return off + jnp.max(pos)

        return lax.fori_loop(0, ECH // 16, step, off)

    lax.fori_loop(0, N_EDGE // ECH, chunk, jnp.int32(0))
    pltpu.sync_copy(slist.at[pl.ds(0, M_MAX)],
                    sp_hbm.at[pl.ds(wid * M_MAX, M_MAX)])
    pltpu.sync_copy(dlist.at[pl.ds(0, M_MAX)],
                    dp_hbm.at[pl.ds(wid * M_MAX, M_MAX)])


def _partition(src, dst):
    return pl.kernel(
        _part_body,
        out_type=(
            jax.ShapeDtypeStruct((NT * M_MAX,), jnp.int32),
            jax.ShapeDtypeStruct((NT * M_MAX,), jnp.int32),
        ),
        mesh=_MESH,
        compiler_params=pltpu.CompilerParams(needs_layout_passes=False),
        scratch_types=[
            pltpu.VMEM((ECH,), jnp.int32),
            pltpu.VMEM((ECH,), jnp.int32),
            pltpu.VMEM((M_MAX + 16,), jnp.int32),
            pltpu.VMEM((M_MAX + 16,), jnp.int32),
        ],
    )(src, dst)


CP = 128


def _edge_body(sp_hbm, dp_hbm, asdt_hbm, xp_hbm, agg_hbm,
               slist, dlist, slisth, alb0, alb1, asrc, adst, denom, acc,
               rowbuf, sem, *, heads):
    wid = _wid()
    base = wid * R
    it16 = jnp.arange(16, dtype=jnp.int32)
    pltpu.sync_copy(sp_hbm.at[pl.ds(wid * M_MAX, M_MAX)], slist)
    pltpu.sync_copy(dp_hbm.at[pl.ds(wid * M_MAX, M_MAX)], dlist)
    pairs = max(heads // 2, 1)

    def pair(p, _):
        for t, alb in enumerate([alb0, alb1][: max(heads // pairs, 1)]):
            h = p * (heads // pairs) + t
            pltpu.sync_copy(asdt_hbm.at[pl.ds(h * N_PAD, N_PAD)], asrc)
            pltpu.sync_copy(asdt_hbm.at[pl.ds((heads + h) * N_PAD + base, R)],
                            adst.at[pl.ds(0, R)])
            adst[pl.ds(R, 16)] = jnp.zeros((16,), jnp.float32)
            _memset(denom, R + 16, 0)

            def p1(j, _):
                sv = slist[pl.ds(j * 16, 16)]
                dlv = dlist[pl.ds(j * 16, 16)]
                if t == 0:
                    slisth[pl.ds(j * 16, 16)] = sv + p * N_PAD
                e = (plsc.load_gather(asrc, [sv])
                     + plsc.load_gather(adst, [dlv]))
                ex = jnp.exp(jnp.maximum(e, 0.2 * e))
                alb[pl.ds(j * 16, 16)] = ex
                plsc.addupdate_scatter(denom, [dlv], ex)
                return 0

            lax.fori_loop(0, M_MAX // 16, p1, 0)

            def rec(i, _):
                denom[pl.ds(i * 16, 16)] = 1.0 / (denom[pl.ds(i * 16, 16)]
                                                  + 1e-16)
                return 0

            lax.fori_loop(0, (R + 16) // 16, rec, 0)

            def p2(j, _):
                dlv = dlist[pl.ds(j * 16, 16)]
                alb[pl.ds(j * 16, 16)] = (alb[pl.ds(j * 16, 16)]
                                          * plsc.load_gather(denom, [dlv]))
                return 0

            lax.fori_loop(0, M_MAX // 16, p2, 0)

        _memset(acc, (R + 1) * CP, 0)
        albb = alb1 if heads > 1 else alb0

        def gchunk(g, _):
            idx = slisth.at[pl.ds(g * GCH, GCH)]
            pltpu.async_copy(xp_hbm.at[idx], rowbuf, sem).wait()

            def edge(k, _):
                jsp = jnp.full((16,), g * GCH + k, jnp.int32)
                alfa = plsc.load_gather(alb0, [jsp])
                alfb = plsc.load_gather(albb, [jsp])
                ib = plsc.load_gather(dlist, [jsp]) * CP
                ksp = jnp.full((16,), k, jnp.int32)
                for q in range(CP // 16):
                    alf = alfa if q < (CP // 32) else alfb
                    val = plsc.load_gather(rowbuf, [ksp, q * 16 + it16])
                    plsc.addupdate_scatter(acc, [ib + (q * 16) + it16],
                                           val * alf)
                return 0

            lax.fori_loop(0, GCH, edge, 0)
            return 0

        lax.fori_loop(0, M_MAX // GCH, gchunk, 0)
        pltpu.sync_copy(acc.at[pl.ds(0, R * CP)],
                        agg_hbm.at[pl.ds((p * N_PAD + base) * CP, R * CP)])
        return 0

    lax.fori_loop(0, pairs, pair, 0)


def _edge_phase(xp_hm, asdt, sp, dp, heads, c):
    pairs = max(heads // 2, 1)
    xp_flat = (xp_hm.reshape(pairs, heads // pairs, N_PAD, c)
               .transpose(0, 2, 1, 3).reshape(pairs * N_PAD, CP))
    asdt = asdt.reshape(2 * heads * N_PAD)
    agg = pl.kernel(
        functools.partial(_edge_body, heads=heads),
        out_type=jax.ShapeDtypeStruct((pairs * N_PAD * CP,), jnp.float32),
        mesh=_MESH,
        compiler_params=pltpu.CompilerParams(needs_layout_passes=False),
        scratch_types=[
            pltpu.VMEM((M_MAX,), jnp.int32),
            pltpu.VMEM((M_MAX,), jnp.int32),
            pltpu.VMEM((M_MAX,), jnp.int32),
            pltpu.VMEM((M_MAX,), jnp.float32),
            pltpu.VMEM((M_MAX,), jnp.float32),
            pltpu.VMEM((N_PAD,), jnp.float32),
            pltpu.VMEM((R + 16,), jnp.float32),
            pltpu.VMEM((R + 16,), jnp.float32),
            pltpu.VMEM(((R + 1) * CP,), jnp.float32),
            pltpu.VMEM((GCH, CP), jnp.float32),
            pltpu.SemaphoreType.DMA,
        ],
    )(sp, dp, asdt, xp_flat)
    return (agg.reshape(pairs, N_PAD, CP).transpose(1, 0, 2)
            .reshape(N_PAD, heads * c))


def kernel(x, edge_index, W1, as1, ad1, b1, W2, as2, ad2, b2, W3, as3, ad3, b3):
    n = x.shape[0]
    src = edge_index[0].astype(jnp.int32)
    dst = edge_index[1].astype(jnp.int32)
    xpad = jnp.pad(x, ((0, N_PAD - n), (0, 0)))

    A1 = jnp.concatenate([_att_mat(as1), _att_mat(ad1)], axis=1)
    A2 = jnp.concatenate([_att_mat(as2), _att_mat(ad2)], axis=1)
    A3 = jnp.concatenate([_att_mat(as3), _att_mat(ad3)], axis=1)

    sp, dp = _partition(src, dst)

    zero_b = jnp.zeros((x.shape[1],), jnp.float32)
    xp1, asdt1 = _project(xpad, zero_b, W1, A1, 8, False)
    agg1 = _edge_phase(xp1, asdt1, sp, dp, 8, 64)

    xp2, asdt2 = _project(agg1, b1, W2, A2, 8, True)
    agg2 = _edge_phase(xp2, asdt2, sp, dp, 8, 64)

    xp3, asdt3 = _project(agg2, b2, W3, A3, 1, True)
    agg3 = _edge_phase(xp3, asdt3, sp, dp, 1, 128)

    return _mean_pool(agg3, b3)


# speedup vs baseline: 7.8756x; 1.2477x over previous
import functools

import jax
import jax.numpy as jnp
from jax import lax
from jax.experimental import pallas as pl
from jax.experimental.pallas import tpu as pltpu
from jax.experimental.pallas import tpu_sc as plsc

N_PAD = 10240
BN = 1024

N_EDGE = 320000
NT = 32
R = 320
M_MAX = 10880
ECH = 2000
GCH = 64

_MESH = plsc.VectorSubcoreMesh(core_axis_name="c", subcore_axis_name="s")


def _proj_kernel(act_ref, b_ref, w_ref, a_ref, xp_ref, asdt_ref, *, heads, relu):
    a = act_ref[...]
    if relu:
        a = jnp.maximum(a + b_ref[...], 0.0)
    xp = jnp.dot(a, w_ref[...], preferred_element_type=jnp.float32,
                 precision=lax.Precision.HIGHEST)
    asd = jnp.dot(xp, a_ref[...], preferred_element_type=jnp.float32,
                  precision=lax.Precision.HIGHEST)
    asdt_ref[...] = asd.T
    c = xp.shape[1] // heads
    xp_ref[...] = xp.reshape(BN, heads, c).transpose(1, 0, 2)


def _project(act, b_prev, W, A_cat, heads, relu):
    f_in = act.shape[1]
    hc = W.shape[1]
    c = hc // heads
    grid = N_PAD // BN
    return pl.pallas_call(
        functools.partial(_proj_kernel, heads=heads, relu=relu),
        grid=(grid,),
        in_specs=[
            pl.BlockSpec((BN, f_in), lambda i: (i, 0)),
            pl.BlockSpec((1, f_in), lambda i: (0, 0)),
            pl.BlockSpec((f_in, hc), lambda i: (0, 0)),
            pl.BlockSpec((hc, 2 * heads), lambda i: (0, 0)),
        ],
        out_specs=[
            pl.BlockSpec((heads, BN, c), lambda i: (0, i, 0)),
            pl.BlockSpec((2 * heads, BN), lambda i: (0, i)),
        ],
        out_shape=[
            jax.ShapeDtypeStruct((heads, N_PAD, c), jnp.float32),
            jax.ShapeDtypeStruct((2 * heads, N_PAD), jnp.float32),
        ],
    )(act, b_prev.reshape(1, f_in), W, A_cat)


def _pool_kernel(h_ref, b_ref, o_ref):
    o_ref[...] = jnp.sum(h_ref[...], axis=0, keepdims=True) / 10000.0 + b_ref[...]


def _mean_pool(h, b3):
    return pl.pallas_call(
        _pool_kernel,
        out_shape=jax.ShapeDtypeStruct((1, h.shape[1]), jnp.float32),
    )(h, b3.reshape(1, h.shape[1]))


def _att_mat(att):
    _, H, C = att.shape
    eye = jnp.eye(H, dtype=jnp.float32)
    return (att[0][:, :, None] * eye[:, None, :]).reshape(H * C, H)


def _wid():
    return lax.axis_index("s") * 2 + lax.axis_index("c")


def _memset(ref, words, value):
    def body(i, _):
        ref[pl.ds(i * 16, 16)] = jnp.full((16,), value, ref.dtype)
        return 0
    lax.fori_loop(0, words // 16, body, 0)


def _part_body(src_hbm, dst_hbm, sp_hbm, dp_hbm, sbuf, dbuf, slist, dlist):
    wid = _wid()
    base = wid * R
    _memset(slist, M_MAX + 16, 0)
    _memset(dlist, M_MAX + 16, R)

    def chunk(k, off):
        pltpu.sync_copy(src_hbm.at[pl.ds(k * ECH, ECH)], sbuf)
        pltpu.sync_copy(dst_hbm.at[pl.ds(k * ECH, ECH)], dbuf)

        def step(j, off):
            sv = sbuf[pl.ds(j * 16, 16)]
            dl = dbuf[pl.ds(j * 16, 16)] - base
            ok = off <= M_MAX - 16
            m = jnp.logical_and((dl >= 0) & (dl < R), ok)
            offc = jnp.minimum(off, M_MAX - 16)
            pos = plsc.cumsum(m.astype(jnp.int32))
            idx = offc + pos - 1
            plsc.store_scatter(slist, [idx], sv, mask=m)
            plsc.store_scatter(dlist, [idx], dl, mask=m)
            return off + jnp.max(pos)

        return lax.fori_loop(0, ECH // 16, step, off)

    lax.fori_loop(0, N_EDGE // ECH, chunk, jnp.int32(0))
    pltpu.sync_copy(slist.at[pl.ds(0, M_MAX)],
                    sp_hbm.at[pl.ds(wid * M_MAX, M_MAX)])
    pltpu.sync_copy(dlist.at[pl.ds(0, M_MAX)],
                    dp_hbm.at[pl.ds(wid * M_MAX, M_MAX)])


def _partition(src, dst):
    return pl.kernel(
        _part_body,
        out_type=(
            jax.ShapeDtypeStruct((NT * M_MAX,), jnp.int32),
            jax.ShapeDtypeStruct((NT * M_MAX,), jnp.int32),
        ),
        mesh=_MESH,
        compiler_params=pltpu.CompilerParams(needs_layout_passes=False),
        scratch_types=[
            pltpu.VMEM((ECH,), jnp.int32),
            pltpu.VMEM((ECH,), jnp.int32),
            pltpu.VMEM((M_MAX + 16,), jnp.int32),
            pltpu.VMEM((M_MAX + 16,), jnp.int32),
        ],
    )(src, dst)


CP = 128


def _edge_body(sp_hbm, dp_hbm, asdt_hbm, xp_hbm, agg_hbm,
               slist, dlist, slisth, alb0, alb1, asrc, adst, denom, acc,
               rowbuf0, rowbuf1, sem0, sem1, *, heads):
    wid = _wid()
    base = wid * R
    it16 = jnp.arange(16, dtype=jnp.int32)
    pltpu.sync_copy(sp_hbm.at[pl.ds(wid * M_MAX, M_MAX)], slist)
    pltpu.sync_copy(dp_hbm.at[pl.ds(wid * M_MAX, M_MAX)], dlist)
    pairs = max(heads // 2, 1)
    NG = M_MAX // GCH

    def pair(p, _):
        for t, alb in enumerate([alb0, alb1][: max(heads // pairs, 1)]):
            h = p * (heads // pairs) + t
            pltpu.sync_copy(asdt_hbm.at[pl.ds(h * N_PAD, N_PAD)], asrc)
            pltpu.sync_copy(asdt_hbm.at[pl.ds((heads + h) * N_PAD + base, R)],
                            adst.at[pl.ds(0, R)])
            adst[pl.ds(R, 16)] = jnp.zeros((16,), jnp.float32)
            _memset(denom, R + 16, 0)

            @plsc.parallel_loop(0, M_MAX // 16, unroll=2)
            def p1(j):
                sv = slist[pl.ds(j * 16, 16)]
                dlv = dlist[pl.ds(j * 16, 16)]
                if t == 0:
                    slisth[pl.ds(j * 16, 16)] = sv + p * N_PAD
                e = (plsc.load_gather(asrc, [sv])
                     + plsc.load_gather(adst, [dlv]))
                ex = jnp.exp(jnp.maximum(e, 0.2 * e))
                alb[pl.ds(j * 16, 16)] = ex
                plsc.addupdate_scatter(denom, [dlv], ex)

            def rec(i, _):
                denom[pl.ds(i * 16, 16)] = 1.0 / (denom[pl.ds(i * 16, 16)]
                                                  + 1e-16)
                return 0

            lax.fori_loop(0, (R + 16) // 16, rec, 0)

            @plsc.parallel_loop(0, M_MAX // 16, unroll=2)
            def p2(j):
                dlv = dlist[pl.ds(j * 16, 16)]
                alb[pl.ds(j * 16, 16)] = (alb[pl.ds(j * 16, 16)]
                                          * plsc.load_gather(denom, [dlv]))

        _memset(acc, (R + 1) * CP, 0)
        albb = alb1 if heads > 1 else alb0

        def issue(gidx, buf, sem):
            idx = slisth.at[pl.ds(gidx * GCH, GCH)]
            pltpu.async_copy(xp_hbm.at[idx], buf, sem)

        def drain(buf, sem):
            pltpu.make_async_copy(xp_hbm.at[pl.ds(0, GCH)], buf, sem).wait()

        def process(g, buf):
            @plsc.parallel_loop(0, GCH, unroll=2)
            def edge(k):
                jsp = jnp.full((16,), g * GCH + k, jnp.int32)
                alfa = plsc.load_gather(alb0, [jsp])
                alfb = plsc.load_gather(albb, [jsp])
                ib = plsc.load_gather(dlist, [jsp]) * CP
                ksp = jnp.full((16,), k, jnp.int32)
                for q in range(CP // 16):
                    alf = alfa if q < (CP // 32) else alfb
                    val = plsc.load_gather(buf, [ksp, q * 16 + it16])
                    plsc.addupdate_scatter(acc, [ib + (q * 16) + it16],
                                           val * alf)

        issue(0, rowbuf0, sem0)
        issue(1, rowbuf1, sem1)

        def gpair(gg, _):
            g0 = gg * 2
            for b, (buf, sem) in enumerate([(rowbuf0, sem0),
                                            (rowbuf1, sem1)]):
                g = g0 + b
                drain(buf, sem)
                process(g, buf)
                issue(jnp.minimum(g + 2, NG - 2 + b), buf, sem)
            return 0

        lax.fori_loop(0, NG // 2, gpair, 0)
        drain(rowbuf0, sem0)
        drain(rowbuf1, sem1)
        pltpu.sync_copy(acc.at[pl.ds(0, R * CP)],
                        agg_hbm.at[pl.ds((p * N_PAD + base) * CP, R * CP)])
        return 0

    lax.fori_loop(0, pairs, pair, 0)


def _edge_phase(xp_hm, asdt, sp, dp, heads, c):
    pairs = max(heads // 2, 1)
    xp_flat = (xp_hm.reshape(pairs, heads // pairs, N_PAD, c)
               .transpose(0, 2, 1, 3).reshape(pairs * N_PAD, CP))
    asdt = asdt.reshape(2 * heads * N_PAD)
    agg = pl.kernel(
        functools.partial(_edge_body, heads=heads),
        out_type=jax.ShapeDtypeStruct((pairs * N_PAD * CP,), jnp.float32),
        mesh=_MESH,
        compiler_params=pltpu.CompilerParams(needs_layout_passes=False),
        scratch_types=[
            pltpu.VMEM((M_MAX,), jnp.int32),
            pltpu.VMEM((M_MAX,), jnp.int32),
            pltpu.VMEM((M_MAX,), jnp.int32),
            pltpu.VMEM((M_MAX,), jnp.float32),
            pltpu.VMEM((M_MAX,), jnp.float32),
            pltpu.VMEM((N_PAD,), jnp.float32),
            pltpu.VMEM((R + 16,), jnp.float32),
            pltpu.VMEM((R + 16,), jnp.float32),
            pltpu.VMEM(((R + 1) * CP,), jnp.float32),
            pltpu.VMEM((GCH, CP), jnp.float32),
            pltpu.VMEM((GCH, CP), jnp.float32),
            pltpu.SemaphoreType.DMA,
            pltpu.SemaphoreType.DMA,
        ],
    )(sp, dp, asdt, xp_flat)
    return (agg.reshape(pairs, N_PAD, CP).transpose(1, 0, 2)
            .reshape(N_PAD, heads * c))


def kernel(x, edge_index, W1, as1, ad1, b1, W2, as2, ad2, b2, W3, as3, ad3, b3):
    n = x.shape[0]
    src = edge_index[0].astype(jnp.int32)
    dst = edge_index[1].astype(jnp.int32)
    xpad = jnp.pad(x, ((0, N_PAD - n), (0, 0)))

    A1 = jnp.concatenate([_att_mat(as1), _att_mat(ad1)], axis=1)
    A2 = jnp.concatenate([_att_mat(as2), _att_mat(ad2)], axis=1)
    A3 = jnp.concatenate([_att_mat(as3), _att_mat(ad3)], axis=1)

    sp, dp = _partition(src, dst)

    zero_b = jnp.zeros((x.shape[1],), jnp.float32)
    xp1, asdt1 = _project(xpad, zero_b, W1, A1, 8, False)
    agg1 = _edge_phase(xp1, asdt1, sp, dp, 8, 64)

    xp2, asdt2 = _project(agg1, b1, W2, A2, 8, True)
    agg2 = _edge_phase(xp2, asdt2, sp, dp, 8, 64)

    xp3, asdt3 = _project(agg2, b2, W3, A3, 1, True)
    agg3 = _edge_phase(xp3, asdt3, sp, dp, 1, 128)

    return _mean_pool(agg3, b3)
